# baseline (speedup 1.0000x reference)
import functools

import jax
import jax.numpy as jnp
from jax import lax
from jax.experimental import pallas as pl
from jax.experimental.pallas import tpu as pltpu
from jax.experimental.pallas import tpu_sc as plsc

_NC = 2
_NS = 16
_NW = _NC * _NS
_CHUNK = 128
_CCHUNK = 128
_QG = 8
_N_ACC = 10240
_H = 128


def _sc_aggregate(e_pad, with_gather, chunk):
    epw = e_pad // _NW
    nchunk = epw // chunk
    rpt = _N_ACC // _NS
    mesh = plsc.VectorSubcoreMesh(core_axis_name="c", subcore_axis_name="s")

    sub = chunk // _QG
    scratch = [
        pltpu.VMEM((nchunk, chunk), jnp.int32),
        pltpu.VMEM((nchunk, chunk), jnp.int32),
        pltpu.VMEM((chunk, _H), jnp.float32),
        pltpu.VMEM((chunk, _H), jnp.float32),
        pltpu.VMEM_SHARED((_N_ACC, _H), jnp.float32),
        pltpu.SemaphoreType.DMA,
        pltpu.SemaphoreType.DMA,
        pltpu.SemaphoreType.DMA,
        pltpu.SemaphoreType.DMA,
    ]

    def _body(tab_hbm, src_hbm, dst_hbm, zero_hbm, out_hbm,
              src_v, dst_v, buf0, buf1, acc_sh, gs0, gs1, ss0, ss1):
        cid = lax.axis_index("c")
        sid = lax.axis_index("s")
        wid = sid * _NC + cid
        r0 = sid * rpt
        wrow = wid * nchunk
        bufs = (buf0, buf1)
        gsem = (gs0, gs1)
        ssem = (ss0, ss1)
        pltpu.sync_copy(zero_hbm.at[pl.ds(r0, rpt)], acc_sh.at[pl.ds(r0, rpt)])
        pltpu.sync_copy(dst_hbm.at[pl.ds(wrow, nchunk)], dst_v)

        if with_gather:
            pltpu.sync_copy(src_hbm.at[pl.ds(wrow, nchunk)], src_v)

            def gath(i, b):
                for q in range(_QG):
                    pltpu.async_copy(
                        tab_hbm.at[src_v.at[i, pl.ds(q * sub, sub)]],
                        bufs[b].at[pl.ds(q * sub, sub)], gsem[b])

            def wait_gath(i, b):
                for q in range(_QG):
                    pltpu.make_async_copy(
                        tab_hbm.at[src_v.at[i, pl.ds(q * sub, sub)]],
                        bufs[b].at[pl.ds(q * sub, sub)], gsem[b]).wait()

            def scat(i, b):
                pltpu.async_copy(bufs[b], acc_sh.at[dst_v.at[i]], ssem[b],
                                 add=True)

            def wait_scat(i, b):
                pltpu.make_async_copy(bufs[b], acc_sh.at[dst_v.at[i]],
                                      ssem[b]).wait()

            gath(0, 0)
            plsc.subcore_barrier()

            def pair(g, carry):
                for b in range(2):
                    i = g * 2 + b
                    wait_gath(i, b)
                    scat(i, b)

                    @pl.when(i >= 1)
                    def _():
                        wait_scat(i - 1, 1 - b)

                    @pl.when(i + 1 < nchunk)
                    def _():
                        gath(i + 1, 1 - b)
                return carry

            lax.fori_loop(0, nchunk // 2, pair, 0)
            wait_scat(nchunk - 1, 1)
        else:
            def orow(r, carry):
                for c in range(_H // 16):
                    buf0[r, pl.ds(c * 16, 16)] = jnp.ones((16,), jnp.float32)
                return carry

            lax.fori_loop(0, chunk, orow, 0)
            plsc.subcore_barrier()

            def body(i, carry):
                pltpu.sync_copy(buf0, acc_sh.at[dst_v.at[i]], add=True)
                return carry

            lax.fori_loop(0, nchunk, body, 0)
        plsc.subcore_barrier()
        pltpu.sync_copy(acc_sh.at[pl.ds(r0, rpt)],
                        out_hbm.at[pl.ds(cid * _N_ACC + r0, rpt)])

    out_type = jax.ShapeDtypeStruct((_NC * _N_ACC, _H), jnp.float32)
    if with_gather:
        @functools.partial(pl.kernel, mesh=mesh, out_type=out_type,
                           scratch_types=scratch)
        def agg(tab_hbm, src_hbm, dst_hbm, zero_hbm, out_hbm, *rest):
            _body(tab_hbm, src_hbm, dst_hbm, zero_hbm, out_hbm, *rest)
    else:
        @functools.partial(pl.kernel, mesh=mesh, out_type=out_type,
                           scratch_types=scratch)
        def agg(dst_hbm, zero_hbm, out_hbm, *rest):
            _body(None, None, dst_hbm, zero_hbm, out_hbm, *rest)

    return agg


def _dinv(c0_ref, c1_ref):
    cnt = c0_ref[...] + c1_ref[...] + 1.0
    return lax.rsqrt(cnt)


def _tc_first(x, w, c0, c1, blk):
    n, d = x.shape
    hn = w.shape[1]

    def body(x_ref, w_ref, c0_ref, c1_ref, o_ref):
        dinv = _dinv(c0_ref, c1_ref)
        o_ref[...] = jnp.dot(x_ref[...] * dinv, w_ref[...],
                             preferred_element_type=jnp.float32)

    return pl.pallas_call(
        body,
        grid=(n // blk,),
        in_specs=[
            pl.BlockSpec((blk, d), lambda i: (i, 0)),
            pl.BlockSpec((d, hn), lambda i: (0, 0)),
            pl.BlockSpec((blk, 1), lambda i: (i, 0)),
            pl.BlockSpec((blk, 1), lambda i: (i, 0)),
        ],
        out_specs=pl.BlockSpec((blk, hn), lambda i: (i, 0)),
        out_shape=jax.ShapeDtypeStruct((n, hn), jnp.float32),
    )(x, w, c0, c1)


def _tc_mid(ra, rb, xs, c0, c1, b, w, blk):
    n, d = xs.shape
    hn = w.shape[1]

    def body(ra_ref, rb_ref, xs_ref, c0_ref, c1_ref, b_ref, w_ref, o_ref):
        dinv = _dinv(c0_ref, c1_ref)
        t = dinv * (ra_ref[...] + rb_ref[...] + xs_ref[...]) + b_ref[...]
        h = jnp.where(t > 0, t, jnp.exp(jnp.minimum(t, 0.0)) - 1.0)
        o_ref[...] = jnp.dot(h * dinv, w_ref[...],
                             preferred_element_type=jnp.float32)

    return pl.pallas_call(
        body,
        grid=(n // blk,),
        in_specs=[
            pl.BlockSpec((blk, d), lambda i: (i, 0)),
            pl.BlockSpec((blk, d), lambda i: (i, 0)),
            pl.BlockSpec((blk, d), lambda i: (i, 0)),
            pl.BlockSpec((blk, 1), lambda i: (i, 0)),
            pl.BlockSpec((blk, 1), lambda i: (i, 0)),
            pl.BlockSpec((1, d), lambda i: (0, 0)),
            pl.BlockSpec((d, hn), lambda i: (0, 0)),
        ],
        out_specs=pl.BlockSpec((blk, hn), lambda i: (i, 0)),
        out_shape=jax.ShapeDtypeStruct((n, hn), jnp.float32),
    )(ra, rb, xs, c0, c1, b, w)


def _tc_final(ra0, rb0, ra1, rb1, x0, x1, c0, c1, b, blk):
    n = x0.shape[0]

    def body(ra0_ref, rb0_ref, ra1_ref, rb1_ref, x0_ref, x1_ref,
             c0_ref, c1_ref, b_ref, o_ref):
        dinv = _dinv(c0_ref, c1_ref)
        t0 = dinv * (ra0_ref[...] + rb0_ref[...] + x0_ref[...]) + b_ref[0:1, 0:1]
        t1 = dinv * (ra1_ref[...] + rb1_ref[...] + x1_ref[...]) + b_ref[0:1, 1:2]
        m = jnp.maximum(t0, t1)
        lse = m + jnp.log(jnp.exp(t0 - m) + jnp.exp(t1 - m))
        o_ref[...] = jnp.concatenate([t0 - lse, t1 - lse], axis=1)

    col = pl.BlockSpec((blk, 1), lambda i: (i, 0))
    return pl.pallas_call(
        body,
        grid=(n // blk,),
        in_specs=[col, col, col, col, col, col, col, col,
                  pl.BlockSpec((1, 2), lambda i: (0, 0))],
        out_specs=pl.BlockSpec((blk, 2), lambda i: (i, 0)),
        out_shape=jax.ShapeDtypeStruct((n, 2), jnp.float32),
    )(ra0, rb0, ra1, rb1, x0, x1, c0, c1, b)


def kernel(x, edge_index, W1, b1, W2, b2, W3, b3):
    x = x.astype(jnp.float32)
    n = x.shape[0]
    e = edge_index.shape[1]
    grain = _NW * _CHUNK
    e_pad = ((e + grain - 1) // grain) * grain
    pad = e_pad - e
    blk = 1000

    src_p = jnp.concatenate(
        [edge_index[0].astype(jnp.int32),
         jnp.zeros((pad,), jnp.int32)]).reshape(-1, _CHUNK)
    dst_p = jnp.concatenate(
        [edge_index[1].astype(jnp.int32),
         jnp.full((pad,), n, jnp.int32)])
    dst_c = dst_p.reshape(-1, _CCHUNK)
    dst_p = dst_p.reshape(-1, _CHUNK)

    z128 = jnp.zeros((_N_ACC, _H), jnp.float32)

    agg = _sc_aggregate(e_pad, True, _CHUNK)

    counts = _sc_aggregate(e_pad, False, _CCHUNK)(dst_c, z128)
    c0 = counts[:n, 0:1]
    c1 = counts[_N_ACC:_N_ACC + n, 0:1]

    xs1 = _tc_first(x, W1, c0, c1, blk)
    raw1 = agg(xs1, src_p, dst_p, z128)
    xs2 = _tc_mid(raw1[:n], raw1[_N_ACC:_N_ACC + n], xs1, c0, c1,
                  b1.reshape(1, -1), W2, blk)
    raw2 = agg(xs2, src_p, dst_p, z128)
    w3p = jnp.pad(W3, ((0, 0), (0, _H - W3.shape[1])))
    xs3 = _tc_mid(raw2[:n], raw2[_N_ACC:_N_ACC + n], xs2, c0, c1,
                  b2.reshape(1, -1), w3p, blk)
    raw3 = agg(xs3, src_p, dst_p, z128)
    return _tc_final(raw3[:n, 0:1], raw3[_N_ACC:_N_ACC + n, 0:1],
                     raw3[:n, 1:2], raw3[_N_ACC:_N_ACC + n, 1:2],
                     xs3[:, 0:1], xs3[:, 1:2], c0, c1,
                     b3.reshape(1, -1), blk)


# speedup vs baseline: 1.0617x; 1.0617x over previous
import functools

import jax
import jax.numpy as jnp
from jax import lax
from jax.experimental import pallas as pl
from jax.experimental.pallas import tpu as pltpu
from jax.experimental.pallas import tpu_sc as plsc

_NC = 2
_NS = 16
_NW = _NC * _NS
_CHUNK = 128
_CCHUNK = 128
_QG = 2
_CPW0 = 56
_CPW1 = 24
_N_ACC = 10240
_H = 128


def _sc_aggregate(e_pad, with_gather, chunk):
    epw = e_pad // _NW
    nchunk = epw // chunk
    rpt = _N_ACC // _NS
    mesh = plsc.VectorSubcoreMesh(core_axis_name="c", subcore_axis_name="s")

    sub = chunk // _QG
    nslab = max(_CPW0, nchunk) if with_gather else nchunk
    scratch = [
        pltpu.VMEM((nslab, chunk), jnp.int32),
        pltpu.VMEM((nslab, chunk), jnp.int32),
        pltpu.VMEM((chunk, _H), jnp.float32),
        pltpu.VMEM((chunk, _H), jnp.float32),
        pltpu.VMEM_SHARED((_N_ACC, _H), jnp.float32),
        pltpu.SemaphoreType.DMA,
        pltpu.SemaphoreType.DMA,
        pltpu.SemaphoreType.DMA,
        pltpu.SemaphoreType.DMA,
    ]

    def _body(tab_hbm, src_hbm, dst_hbm, zero_hbm, out_hbm,
              src_v, dst_v, buf0, buf1, acc_sh, gs0, gs1, ss0, ss1):
        cid = lax.axis_index("c")
        sid = lax.axis_index("s")
        wid = sid * _NC + cid
        r0 = sid * rpt
        bufs = (buf0, buf1)
        gsem = (gs0, gs1)
        ssem = (ss0, ss1)
        pltpu.sync_copy(zero_hbm.at[pl.ds(r0, rpt)], acc_sh.at[pl.ds(r0, rpt)])

        if with_gather:
            nc = jnp.where(cid == 0, _CPW0, _CPW1)
            wrow = jnp.where(cid == 0, sid * _CPW0,
                             _NS * _CPW0 + sid * _CPW1)

            @pl.when(cid == 0)
            def _():
                pltpu.sync_copy(dst_hbm.at[pl.ds(wrow, _CPW0)],
                                dst_v.at[pl.ds(0, _CPW0)])
                pltpu.sync_copy(src_hbm.at[pl.ds(wrow, _CPW0)],
                                src_v.at[pl.ds(0, _CPW0)])

            @pl.when(cid == 1)
            def _():
                pltpu.sync_copy(dst_hbm.at[pl.ds(wrow, _CPW1)],
                                dst_v.at[pl.ds(0, _CPW1)])
                pltpu.sync_copy(src_hbm.at[pl.ds(wrow, _CPW1)],
                                src_v.at[pl.ds(0, _CPW1)])

            def gath(i, b):
                for q in range(_QG):
                    pltpu.async_copy(
                        tab_hbm.at[src_v.at[i, pl.ds(q * sub, sub)]],
                        bufs[b].at[pl.ds(q * sub, sub)], gsem[b])

            def wait_gath(i, b):
                for q in range(_QG):
                    pltpu.make_async_copy(
                        tab_hbm.at[src_v.at[i, pl.ds(q * sub, sub)]],
                        bufs[b].at[pl.ds(q * sub, sub)], gsem[b]).wait()

            def scat(i, b):
                pltpu.async_copy(bufs[b], acc_sh.at[dst_v.at[i]], ssem[b],
                                 add=True)

            def wait_scat(i, b):
                pltpu.make_async_copy(bufs[b], acc_sh.at[dst_v.at[i]],
                                      ssem[b]).wait()

            gath(0, 0)
            plsc.subcore_barrier()

            def pair(g, carry):
                for b in range(2):
                    i = g * 2 + b
                    wait_gath(i, b)
                    scat(i, b)

                    @pl.when(i >= 1)
                    def _():
                        wait_scat(i - 1, 1 - b)

                    @pl.when(i + 1 < nc)
                    def _():
                        gath(i + 1, 1 - b)
                return carry

            lax.fori_loop(0, nc // 2, pair, 0)
            wait_scat(nc - 1, 1)
        else:
            wrow = wid * nchunk
            pltpu.sync_copy(dst_hbm.at[pl.ds(wrow, nchunk)], dst_v)
            def orow(r, carry):
                for c in range(_H // 16):
                    buf0[r, pl.ds(c * 16, 16)] = jnp.ones((16,), jnp.float32)
                return carry

            lax.fori_loop(0, chunk, orow, 0)
            plsc.subcore_barrier()

            def body(i, carry):
                pltpu.sync_copy(buf0, acc_sh.at[dst_v.at[i]], add=True)
                return carry

            lax.fori_loop(0, nchunk, body, 0)
        plsc.subcore_barrier()
        pltpu.sync_copy(acc_sh.at[pl.ds(r0, rpt)],
                        out_hbm.at[pl.ds(cid * _N_ACC + r0, rpt)])

    out_type = jax.ShapeDtypeStruct((_NC * _N_ACC, _H), jnp.float32)
    if with_gather:
        @functools.partial(pl.kernel, mesh=mesh, out_type=out_type,
                           scratch_types=scratch)
        def agg(tab_hbm, src_hbm, dst_hbm, zero_hbm, out_hbm, *rest):
            _body(tab_hbm, src_hbm, dst_hbm, zero_hbm, out_hbm, *rest)
    else:
        @functools.partial(pl.kernel, mesh=mesh, out_type=out_type,
                           scratch_types=scratch)
        def agg(dst_hbm, zero_hbm, out_hbm, *rest):
            _body(None, None, dst_hbm, zero_hbm, out_hbm, *rest)

    return agg


def _dinv(c0_ref, c1_ref):
    cnt = c0_ref[...] + c1_ref[...] + 1.0
    return lax.rsqrt(cnt)


def _tc_first(x, w, c0, c1, blk):
    n, d = x.shape
    hn = w.shape[1]

    def body(x_ref, w_ref, c0_ref, c1_ref, o_ref):
        dinv = _dinv(c0_ref, c1_ref)
        o_ref[...] = jnp.dot(x_ref[...] * dinv, w_ref[...],
                             preferred_element_type=jnp.float32)

    return pl.pallas_call(
        body,
        grid=(n // blk,),
        in_specs=[
            pl.BlockSpec((blk, d), lambda i: (i, 0)),
            pl.BlockSpec((d, hn), lambda i: (0, 0)),
            pl.BlockSpec((blk, 1), lambda i: (i, 0)),
            pl.BlockSpec((blk, 1), lambda i: (i, 0)),
        ],
        out_specs=pl.BlockSpec((blk, hn), lambda i: (i, 0)),
        out_shape=jax.ShapeDtypeStruct((n, hn), jnp.float32),
    )(x, w, c0, c1)


def _tc_mid(ra, rb, xs, c0, c1, b, w, blk):
    n, d = xs.shape
    hn = w.shape[1]

    def body(ra_ref, rb_ref, xs_ref, c0_ref, c1_ref, b_ref, w_ref, o_ref):
        dinv = _dinv(c0_ref, c1_ref)
        t = dinv * (ra_ref[...] + rb_ref[...] + xs_ref[...]) + b_ref[...]
        h = jnp.where(t > 0, t, jnp.exp(jnp.minimum(t, 0.0)) - 1.0)
        o_ref[...] = jnp.dot(h * dinv, w_ref[...],
                             preferred_element_type=jnp.float32)

    return pl.pallas_call(
        body,
        grid=(n // blk,),
        in_specs=[
            pl.BlockSpec((blk, d), lambda i: (i, 0)),
            pl.BlockSpec((blk, d), lambda i: (i, 0)),
            pl.BlockSpec((blk, d), lambda i: (i, 0)),
            pl.BlockSpec((blk, 1), lambda i: (i, 0)),
            pl.BlockSpec((blk, 1), lambda i: (i, 0)),
            pl.BlockSpec((1, d), lambda i: (0, 0)),
            pl.BlockSpec((d, hn), lambda i: (0, 0)),
        ],
        out_specs=pl.BlockSpec((blk, hn), lambda i: (i, 0)),
        out_shape=jax.ShapeDtypeStruct((n, hn), jnp.float32),
    )(ra, rb, xs, c0, c1, b, w)


def _tc_final(ra0, rb0, ra1, rb1, x0, x1, c0, c1, b, blk):
    n = x0.shape[0]

    def body(ra0_ref, rb0_ref, ra1_ref, rb1_ref, x0_ref, x1_ref,
             c0_ref, c1_ref, b_ref, o_ref):
        dinv = _dinv(c0_ref, c1_ref)
        t0 = dinv * (ra0_ref[...] + rb0_ref[...] + x0_ref[...]) + b_ref[0:1, 0:1]
        t1 = dinv * (ra1_ref[...] + rb1_ref[...] + x1_ref[...]) + b_ref[0:1, 1:2]
        m = jnp.maximum(t0, t1)
        lse = m + jnp.log(jnp.exp(t0 - m) + jnp.exp(t1 - m))
        o_ref[...] = jnp.concatenate([t0 - lse, t1 - lse], axis=1)

    col = pl.BlockSpec((blk, 1), lambda i: (i, 0))
    return pl.pallas_call(
        body,
        grid=(n // blk,),
        in_specs=[col, col, col, col, col, col, col, col,
                  pl.BlockSpec((1, 2), lambda i: (0, 0))],
        out_specs=pl.BlockSpec((blk, 2), lambda i: (i, 0)),
        out_shape=jax.ShapeDtypeStruct((n, 2), jnp.float32),
    )(ra0, rb0, ra1, rb1, x0, x1, c0, c1, b)


def kernel(x, edge_index, W1, b1, W2, b2, W3, b3):
    x = x.astype(jnp.float32)
    n = x.shape[0]
    e = edge_index.shape[1]
    grain = _NW * _CHUNK
    e_pad = ((e + grain - 1) // grain) * grain
    pad = e_pad - e
    blk = 1000

    src_p = jnp.concatenate(
        [edge_index[0].astype(jnp.int32),
         jnp.zeros((pad,), jnp.int32)]).reshape(-1, _CHUNK)
    dst_p = jnp.concatenate(
        [edge_index[1].astype(jnp.int32),
         jnp.full((pad,), n, jnp.int32)])
    dst_c = dst_p.reshape(-1, _CCHUNK)
    dst_p = dst_p.reshape(-1, _CHUNK)

    z128 = jnp.zeros((_N_ACC, _H), jnp.float32)

    agg = _sc_aggregate(e_pad, True, _CHUNK)

    counts = _sc_aggregate(e_pad, False, _CCHUNK)(dst_c, z128)
    c0 = counts[:n, 0:1]
    c1 = counts[_N_ACC:_N_ACC + n, 0:1]

    xs1 = _tc_first(x, W1, c0, c1, blk)
    raw1 = agg(xs1, src_p, dst_p, z128)
    xs2 = _tc_mid(raw1[:n], raw1[_N_ACC:_N_ACC + n], xs1, c0, c1,
                  b1.reshape(1, -1), W2, blk)
    raw2 = agg(xs2, src_p, dst_p, z128)
    w3p = jnp.pad(W3, ((0, 0), (0, _H - W3.shape[1])))
    xs3 = _tc_mid(raw2[:n], raw2[_N_ACC:_N_ACC + n], xs2, c0, c1,
                  b2.reshape(1, -1), w3p, blk)
    raw3 = agg(xs3, src_p, dst_p, z128)
    return _tc_final(raw3[:n, 0:1], raw3[_N_ACC:_N_ACC + n, 0:1],
                     raw3[:n, 1:2], raw3[_N_ACC:_N_ACC + n, 1:2],
                     xs3[:, 0:1], xs3[:, 1:2], c0, c1,
                     b3.reshape(1, -1), blk)


# speedup vs baseline: 1.0828x; 1.0198x over previous
import functools

import jax
import jax.numpy as jnp
from jax import lax
from jax.experimental import pallas as pl
from jax.experimental.pallas import tpu as pltpu
from jax.experimental.pallas import tpu_sc as plsc

_NC = 2
_NS = 16
_NW = _NC * _NS
_CHUNK = 128
_CCHUNK = 128
_GC = 32
_K = 8
_LOOK = 4
_CPW0 = 56
_CPW1 = 24
_N_ACC = 10240
_H = 128


def _sc_aggregate(e_pad, with_gather, chunk):
    epw = e_pad // _NW
    nchunk = epw // chunk
    rpt = _N_ACC // _NS
    mesh = plsc.VectorSubcoreMesh(core_axis_name="c", subcore_axis_name="s")

    nslab = max(_CPW0, nchunk) if with_gather else nchunk
    scratch = [
        pltpu.VMEM((nslab, chunk), jnp.int32),
        pltpu.VMEM((nslab, chunk), jnp.int32),
    ]
    if with_gather:
        scratch += [pltpu.VMEM((_GC, _H), jnp.float32) for _ in range(_K)]
        scratch += [pltpu.VMEM((_GC,), jnp.int32) for _ in range(_K)]
        scratch += [pltpu.VMEM_SHARED((_N_ACC, _H), jnp.float32)]
        scratch += [pltpu.SemaphoreType.DMA for _ in range(2 * _K)]
    else:
        scratch += [pltpu.VMEM((chunk, _H), jnp.float32),
                    pltpu.VMEM_SHARED((_N_ACC, _H), jnp.float32)]

    def _body(tab_hbm, src_hbm, dst_hbm, zero_hbm, out_hbm, src_v, dst_v,
              bufs, idx1, acc_sh, gsem, ssem):
        cid = lax.axis_index("c")
        sid = lax.axis_index("s")
        wid = sid * _NC + cid
        r0 = sid * rpt
        pltpu.sync_copy(zero_hbm.at[pl.ds(r0, rpt)], acc_sh.at[pl.ds(r0, rpt)])

        if with_gather:
            rows = jnp.where(cid == 0, _CPW0, _CPW1)
            nc = rows * (chunk // _GC)
            wrow = jnp.where(cid == 0, sid * _CPW0,
                             _NS * _CPW0 + sid * _CPW1)

            @pl.when(cid == 0)
            def _():
                pltpu.sync_copy(dst_hbm.at[pl.ds(wrow, _CPW0)],
                                dst_v.at[pl.ds(0, _CPW0)])
                pltpu.sync_copy(src_hbm.at[pl.ds(wrow, _CPW0)],
                                src_v.at[pl.ds(0, _CPW0)])

            @pl.when(cid == 1)
            def _():
                pltpu.sync_copy(dst_hbm.at[pl.ds(wrow, _CPW1)],
                                dst_v.at[pl.ds(0, _CPW1)])
                pltpu.sync_copy(src_hbm.at[pl.ds(wrow, _CPW1)],
                                src_v.at[pl.ds(0, _CPW1)])

            def sidx(i):
                return src_v.at[i // 4, pl.ds((i % 4) * _GC, _GC)]

            def gath(i, b):
                pltpu.async_copy(tab_hbm.at[sidx(i)], bufs[b], gsem[b])

            def wait_gath(i, b):
                pltpu.make_async_copy(tab_hbm.at[sidx(i)], bufs[b],
                                      gsem[b]).wait()

            def idxcopy(i, b):
                r = i // 4
                c0 = (i % 4) * _GC
                idx1[b][pl.ds(0, 16)] = dst_v[r, pl.ds(c0, 16)]
                idx1[b][pl.ds(16, 16)] = dst_v[r, pl.ds(c0 + 16, 16)]

            def scat(i, b):
                pltpu.async_copy(bufs[b], acc_sh.at[idx1[b]], ssem[b],
                                 add=True)

            def wait_scat(b):
                pltpu.make_async_copy(bufs[b], acc_sh.at[idx1[b]],
                                      ssem[b]).wait()

            for s in range(_LOOK):
                gath(s, s)
            plsc.subcore_barrier()

            def group(g, carry):
                for b in range(_K):
                    i = g * _K + b
                    bn = (b + _LOOK) % _K
                    wait_gath(i, b)
                    idxcopy(i, b)
                    scat(i, b)
                    if b >= _LOOK:
                        wait_scat(bn)
                        gath(i + _LOOK, bn)
                    else:
                        @pl.when(g > 0)
                        def _():
                            wait_scat(bn)
                        gath(i + _LOOK, bn)
                return carry

            lax.fori_loop(0, nc // _K - 1, group, 0)
            for b in range(_LOOK):
                i = nc - _K + b
                bn = (b + _LOOK) % _K
                wait_gath(i, b)
                idxcopy(i, b)
                scat(i, b)
                wait_scat(bn)
                gath(i + _LOOK, bn)
            for b in range(_LOOK, _K):
                i = nc - _K + b
                wait_gath(i, b)
                idxcopy(i, b)
                pltpu.sync_copy(bufs[b], acc_sh.at[idx1[b]], add=True)
            for b in range(_LOOK):
                wait_scat(b)
        else:
            wrow = wid * nchunk
            pltpu.sync_copy(dst_hbm.at[pl.ds(wrow, nchunk)], dst_v)

            def orow(r, carry):
                for c in range(_H // 16):
                    bufs[r, pl.ds(c * 16, 16)] = jnp.ones((16,), jnp.float32)
                return carry

            lax.fori_loop(0, chunk, orow, 0)
            plsc.subcore_barrier()

            def body(i, carry):
                pltpu.sync_copy(bufs, acc_sh.at[dst_v.at[i]], add=True)
                return carry

            lax.fori_loop(0, nchunk, body, 0)
        plsc.subcore_barrier()
        pltpu.sync_copy(acc_sh.at[pl.ds(r0, rpt)],
                        out_hbm.at[pl.ds(cid * _N_ACC + r0, rpt)])

    out_type = jax.ShapeDtypeStruct((_NC * _N_ACC, _H), jnp.float32)
    if with_gather:
        @functools.partial(pl.kernel, mesh=mesh, out_type=out_type,
                           scratch_types=scratch)
        def agg(tab_hbm, src_hbm, dst_hbm, zero_hbm, out_hbm,
                src_v, dst_v, *rest):
            bufs = rest[:_K]
            idx1 = rest[_K:2 * _K]
            acc_sh = rest[2 * _K]
            gsem = rest[2 * _K + 1:3 * _K + 1]
            ssem = rest[3 * _K + 1:]
            _body(tab_hbm, src_hbm, dst_hbm, zero_hbm, out_hbm,
                  src_v, dst_v, bufs, idx1, acc_sh, gsem, ssem)
    else:
        @functools.partial(pl.kernel, mesh=mesh, out_type=out_type,
                           scratch_types=scratch)
        def agg(dst_hbm, zero_hbm, out_hbm, src_v, dst_v, ones_v, acc_sh):
            _body(None, None, dst_hbm, zero_hbm, out_hbm,
                  src_v, dst_v, ones_v, None, acc_sh, None, None)

    return agg


def _dinv(c0_ref, c1_ref):
    cnt = c0_ref[...] + c1_ref[...] + 1.0
    return lax.rsqrt(cnt)


def _tc_first(x, w, c0, c1, blk):
    n, d = x.shape
    hn = w.shape[1]

    def body(x_ref, w_ref, c0_ref, c1_ref, o_ref):
        dinv = _dinv(c0_ref, c1_ref)
        o_ref[...] = jnp.dot(x_ref[...] * dinv, w_ref[...],
                             preferred_element_type=jnp.float32)

    return pl.pallas_call(
        body,
        grid=(n // blk,),
        in_specs=[
            pl.BlockSpec((blk, d), lambda i: (i, 0)),
            pl.BlockSpec((d, hn), lambda i: (0, 0)),
            pl.BlockSpec((blk, 1), lambda i: (i, 0)),
            pl.BlockSpec((blk, 1), lambda i: (i, 0)),
        ],
        out_specs=pl.BlockSpec((blk, hn), lambda i: (i, 0)),
        out_shape=jax.ShapeDtypeStruct((n, hn), jnp.float32),
    )(x, w, c0, c1)


def _tc_mid(ra, rb, xs, c0, c1, b, w, blk):
    n, d = xs.shape
    hn = w.shape[1]

    def body(ra_ref, rb_ref, xs_ref, c0_ref, c1_ref, b_ref, w_ref, o_ref):
        dinv = _dinv(c0_ref, c1_ref)
        t = dinv * (ra_ref[...] + rb_ref[...] + xs_ref[...]) + b_ref[...]
        h = jnp.where(t > 0, t, jnp.exp(jnp.minimum(t, 0.0)) - 1.0)
        o_ref[...] = jnp.dot(h * dinv, w_ref[...],
                             preferred_element_type=jnp.float32)

    return pl.pallas_call(
        body,
        grid=(n // blk,),
        in_specs=[
            pl.BlockSpec((blk, d), lambda i: (i, 0)),
            pl.BlockSpec((blk, d), lambda i: (i, 0)),
            pl.BlockSpec((blk, d), lambda i: (i, 0)),
            pl.BlockSpec((blk, 1), lambda i: (i, 0)),
            pl.BlockSpec((blk, 1), lambda i: (i, 0)),
            pl.BlockSpec((1, d), lambda i: (0, 0)),
            pl.BlockSpec((d, hn), lambda i: (0, 0)),
        ],
        out_specs=pl.BlockSpec((blk, hn), lambda i: (i, 0)),
        out_shape=jax.ShapeDtypeStruct((n, hn), jnp.float32),
    )(ra, rb, xs, c0, c1, b, w)


def _tc_final(ra0, rb0, ra1, rb1, x0, x1, c0, c1, b, blk):
    n = x0.shape[0]

    def body(ra0_ref, rb0_ref, ra1_ref, rb1_ref, x0_ref, x1_ref,
             c0_ref, c1_ref, b_ref, o_ref):
        dinv = _dinv(c0_ref, c1_ref)
        t0 = dinv * (ra0_ref[...] + rb0_ref[...] + x0_ref[...]) + b_ref[0:1, 0:1]
        t1 = dinv * (ra1_ref[...] + rb1_ref[...] + x1_ref[...]) + b_ref[0:1, 1:2]
        m = jnp.maximum(t0, t1)
        lse = m + jnp.log(jnp.exp(t0 - m) + jnp.exp(t1 - m))
        o_ref[...] = jnp.concatenate([t0 - lse, t1 - lse], axis=1)

    col = pl.BlockSpec((blk, 1), lambda i: (i, 0))
    return pl.pallas_call(
        body,
        grid=(n // blk,),
        in_specs=[col, col, col, col, col, col, col, col,
                  pl.BlockSpec((1, 2), lambda i: (0, 0))],
        out_specs=pl.BlockSpec((blk, 2), lambda i: (i, 0)),
        out_shape=jax.ShapeDtypeStruct((n, 2), jnp.float32),
    )(ra0, rb0, ra1, rb1, x0, x1, c0, c1, b)


def kernel(x, edge_index, W1, b1, W2, b2, W3, b3):
    x = x.astype(jnp.float32)
    n = x.shape[0]
    e = edge_index.shape[1]
    grain = _NW * _CHUNK
    e_pad = ((e + grain - 1) // grain) * grain
    pad = e_pad - e
    blk = 1000

    src_p = jnp.concatenate(
        [edge_index[0].astype(jnp.int32),
         jnp.zeros((pad,), jnp.int32)]).reshape(-1, _CHUNK)
    dst_p = jnp.concatenate(
        [edge_index[1].astype(jnp.int32),
         jnp.full((pad,), n, jnp.int32)])
    dst_c = dst_p.reshape(-1, _CCHUNK)
    dst_p = dst_p.reshape(-1, _CHUNK)

    z128 = jnp.zeros((_N_ACC, _H), jnp.float32)

    agg = _sc_aggregate(e_pad, True, _CHUNK)

    counts = _sc_aggregate(e_pad, False, _CCHUNK)(dst_c, z128)
    c0 = counts[:n, 0:1]
    c1 = counts[_N_ACC:_N_ACC + n, 0:1]

    xs1 = _tc_first(x, W1, c0, c1, blk)
    raw1 = agg(xs1, src_p, dst_p, z128)
    xs2 = _tc_mid(raw1[:n], raw1[_N_ACC:_N_ACC + n], xs1, c0, c1,
                  b1.reshape(1, -1), W2, blk)
    raw2 = agg(xs2, src_p, dst_p, z128)
    w3p = jnp.pad(W3, ((0, 0), (0, _H - W3.shape[1])))
    xs3 = _tc_mid(raw2[:n], raw2[_N_ACC:_N_ACC + n], xs2, c0, c1,
                  b2.reshape(1, -1), w3p, blk)
    raw3 = agg(xs3, src_p, dst_p, z128)
    return _tc_final(raw3[:n, 0:1], raw3[_N_ACC:_N_ACC + n, 0:1],
                     raw3[:n, 1:2], raw3[_N_ACC:_N_ACC + n, 1:2],
                     xs3[:, 0:1], xs3[:, 1:2], c0, c1,
                     b3.reshape(1, -1), blk)
